# baseline (speedup 1.0000x reference)
import functools

import jax
import jax.numpy as jnp
from jax import lax
from jax.experimental import pallas as pl
from jax.experimental.pallas import tpu as pltpu
from jax.experimental.pallas import tpu_sc as plsc

HIDDEN = 1024
FFN = 768
E = 8
T = 2048
BLK = 256
NT = (2 * T) // BLK + E - 1
PAD_ROWS = NT * BLK

NC = 2
NS = 16
NW = NC * NS
TOK_PER_W = T // NW
CH = 32
CCH = 256
NCH = T // CCH


def _routing_body(x_ref, gwt_ref, pos1_ref, pos2_ref, w1_ref, w2_ref, eid_ref,
                  xb_ref):
    x = x_ref[...]
    xb_ref.bitcast(jnp.bfloat16)[...] = x.astype(jnp.bfloat16).reshape(2 * T, HIDDEN // 2)
    logits = jnp.dot(x, gwt_ref[...], preferred_element_type=jnp.float32)
    l1 = jnp.max(logits, axis=-1, keepdims=True)
    i1 = jnp.argmax(logits, axis=-1)[:, None]
    cols = jax.lax.broadcasted_iota(jnp.int32, logits.shape, 1)
    l_m = jnp.where(cols == i1, -jnp.inf, logits)
    l2 = jnp.max(l_m, axis=-1, keepdims=True)
    i2 = jnp.argmax(l_m, axis=-1)[:, None]
    w1 = 1.0 / (1.0 + jnp.exp(l2 - l1))
    w2 = 1.0 - w1

    oh1 = cols == i1
    oh2 = cols == i2
    a = (oh1 | oh2).astype(jnp.bfloat16)

    r_i = jax.lax.broadcasted_iota(jnp.int32, (CCH, CCH), 0)
    c_i = jax.lax.broadcasted_iota(jnp.int32, (CCH, CCH), 1)
    tri_c = (c_i < r_i).astype(jnp.bfloat16)
    a3 = a.reshape(NCH, CCH, E)
    within = [
        jnp.dot(tri_c, a3[c], preferred_element_type=jnp.float32)
        for c in range(NCH)
    ]
    s = jnp.stack(
        [jnp.sum(a3[c].astype(jnp.float32), axis=0) for c in range(NCH)], axis=0
    )
    ch_r = jax.lax.broadcasted_iota(jnp.int32, (NCH, NCH), 0)
    ch_c = jax.lax.broadcasted_iota(jnp.int32, (NCH, NCH), 1)
    tri_ch = (ch_c < ch_r).astype(jnp.float32)
    excl_ch = jnp.dot(tri_ch, s, preferred_element_type=jnp.float32)
    p_excl = jnp.concatenate(
        [within[c] + excl_ch[c:c + 1, :] for c in range(NCH)], axis=0
    )

    cnt = jnp.sum(s, axis=0, keepdims=True)
    rup = jnp.ceil(cnt / BLK) * BLK
    e_r = jax.lax.broadcasted_iota(jnp.int32, (E, E), 0)
    e_c = jax.lax.broadcasted_iota(jnp.int32, (E, E), 1)
    tri8 = (e_r < e_c).astype(jnp.float32)
    astart = jnp.dot(rup, tri8, preferred_element_type=jnp.float32)

    posf = astart + p_excl
    pos1 = jnp.sum(jnp.where(oh1, posf, 0.0), axis=-1, keepdims=True)
    pos2 = jnp.sum(jnp.where(oh2, posf, 0.0), axis=-1, keepdims=True)
    pos1_ref[...] = pos1.astype(jnp.int32)
    pos2_ref[...] = pos2.astype(jnp.int32)
    w1_ref[...] = jnp.broadcast_to(w1, (T, 16))
    w2_ref[...] = jnp.broadcast_to(w2, (T, 16))

    cum_incl = astart + rup
    tile_start = jax.lax.broadcasted_iota(jnp.int32, (32, E), 0).astype(jnp.float32) * BLK
    eid = jnp.sum((cum_incl <= tile_start).astype(jnp.int32), axis=-1)
    n_active = (jnp.sum(rup) / BLK).astype(jnp.int32)
    e_iota = jax.lax.broadcasted_iota(jnp.int32, (1, E), 1)
    last_e = jnp.max(jnp.where(rup > 0, e_iota, 0))
    idx32 = jax.lax.broadcasted_iota(jnp.int32, (32,), 0)
    eid_ref[...] = jnp.where(idx32 == NT, n_active, jnp.minimum(eid, last_e))


def _routing(x, gate_wt):
    return pl.pallas_call(
        _routing_body,
        out_shape=(
            jax.ShapeDtypeStruct((T, 1), jnp.int32),
            jax.ShapeDtypeStruct((T, 1), jnp.int32),
            jax.ShapeDtypeStruct((T, 16), jnp.float32),
            jax.ShapeDtypeStruct((T, 16), jnp.float32),
            jax.ShapeDtypeStruct((32,), jnp.int32),
            jax.ShapeDtypeStruct((T, HIDDEN // 2), jnp.int32),
        ),
    )(x, gate_wt)


def _dispatch_body(xb_hbm, p1_hbm, p2_hbm, xs_hbm, xv, i1v, i2v, s1, s2):
    wid = lax.axis_index("s") * NC + lax.axis_index("c")
    base = wid * TOK_PER_W
    pltpu.sync_copy(xb_hbm.at[pl.ds(base, TOK_PER_W)], xv)
    pltpu.sync_copy(p1_hbm.at[pl.ds(base, TOK_PER_W)], i1v)
    pltpu.sync_copy(p2_hbm.at[pl.ds(base, TOK_PER_W)], i2v)
    c1 = pltpu.async_copy(xv, xs_hbm.at[i1v], s1)
    c2 = pltpu.async_copy(xv, xs_hbm.at[i2v], s2)
    c1.wait()
    c2.wait()


def _dispatch(xb, pos1, pos2):
    mesh = plsc.VectorSubcoreMesh(core_axis_name="c", subcore_axis_name="s")
    f = pl.kernel(
        _dispatch_body,
        out_type=jax.ShapeDtypeStruct((PAD_ROWS, HIDDEN // 2), jnp.int32),
        mesh=mesh,
        scratch_types=[
            pltpu.VMEM((TOK_PER_W, HIDDEN // 2), jnp.int32),
            pltpu.VMEM((TOK_PER_W,), jnp.int32),
            pltpu.VMEM((TOK_PER_W,), jnp.int32),
            pltpu.SemaphoreType.DMA,
            pltpu.SemaphoreType.DMA,
        ],
    )
    return f(xb, pos1, pos2)


def _gmm_body(eids_ref, xs_ref, g_ref, u_ref, d_ref, out_ref, gb_ref, ub_ref, db_ref):
    i = pl.program_id(0)
    n_active = eids_ref[NT]
    changed = jnp.logical_or(i == 0, eids_ref[i] != eids_ref[jnp.maximum(i - 1, 0)])

    @pl.when(jnp.logical_and(i < n_active, changed))
    def _():
        gb_ref[...] = g_ref[0].astype(jnp.bfloat16)
        ub_ref[...] = u_ref[0].astype(jnp.bfloat16)
        db_ref[...] = d_ref[0].astype(jnp.bfloat16)

    @pl.when(i < n_active)
    def _():
        x = xs_ref.bitcast(jnp.bfloat16)[...].reshape(BLK, HIDDEN)
        g = jnp.dot(x, gb_ref[...], preferred_element_type=jnp.float32)
        u = jnp.dot(x, ub_ref[...], preferred_element_type=jnp.float32)
        act = ((g * jax.nn.sigmoid(g)) * u).astype(jnp.bfloat16)
        y = jnp.dot(act, db_ref[...], preferred_element_type=jnp.float32)
        out_ref.bitcast(jnp.bfloat16)[...] = y.astype(jnp.bfloat16).reshape(2 * BLK, HIDDEN // 2)


def _gmm(eids, xs, gate_proj, up_proj, down_proj):
    grid_spec = pltpu.PrefetchScalarGridSpec(
        num_scalar_prefetch=1,
        grid=(NT,),
        in_specs=[
            pl.BlockSpec((BLK, HIDDEN // 2),
                         lambda i, eids: (jnp.minimum(i, eids[NT] - 1), 0)),
            pl.BlockSpec((1, HIDDEN, FFN), lambda i, eids: (eids[i], 0, 0)),
            pl.BlockSpec((1, HIDDEN, FFN), lambda i, eids: (eids[i], 0, 0)),
            pl.BlockSpec((1, FFN, HIDDEN), lambda i, eids: (eids[i], 0, 0)),
        ],
        out_specs=pl.BlockSpec((BLK, HIDDEN // 2),
                               lambda i, eids: (jnp.minimum(i, eids[NT] - 1), 0)),
        scratch_shapes=[
            pltpu.VMEM((HIDDEN, FFN), jnp.bfloat16),
            pltpu.VMEM((HIDDEN, FFN), jnp.bfloat16),
            pltpu.VMEM((FFN, HIDDEN), jnp.bfloat16),
        ],
    )
    return pl.pallas_call(
        _gmm_body,
        grid_spec=grid_spec,
        out_shape=jax.ShapeDtypeStruct((PAD_ROWS, HIDDEN // 2), jnp.int32),
        compiler_params=pltpu.CompilerParams(
            dimension_semantics=("arbitrary",),
        ),
    )(eids, xs, gate_proj, up_proj, down_proj)


def _combine_body(ys_hbm, p1_hbm, p2_hbm, y1_hbm, y2_hbm, i1v, i2v,
                  b10, b11, b20, b21, s10, s11, s20, s21):
    wid = lax.axis_index("s") * NC + lax.axis_index("c")
    base = wid * TOK_PER_W
    pltpu.sync_copy(p1_hbm.at[pl.ds(base, TOK_PER_W)], i1v)
    pltpu.sync_copy(p2_hbm.at[pl.ds(base, TOK_PER_W)], i2v)
    bufs1, bufs2 = (b10, b11), (b20, b21)
    sems1, sems2 = (s10, s11), (s20, s21)
    cps = []
    for c in range(TOK_PER_W // CH):
        cps.append((
            pltpu.async_copy(ys_hbm.at[i1v.at[pl.ds(c * CH, CH)]], bufs1[c], sems1[c]),
            pltpu.async_copy(ys_hbm.at[i2v.at[pl.ds(c * CH, CH)]], bufs2[c], sems2[c]),
        ))
    for c in range(TOK_PER_W // CH):
        g1, g2 = cps[c]
        g1.wait()
        pltpu.sync_copy(bufs1[c], y1_hbm.at[pl.ds(base + c * CH, CH)])
        g2.wait()
        pltpu.sync_copy(bufs2[c], y2_hbm.at[pl.ds(base + c * CH, CH)])


def _combine(ys, pos1, pos2):
    mesh = plsc.VectorSubcoreMesh(core_axis_name="c", subcore_axis_name="s")
    f = pl.kernel(
        _combine_body,
        out_type=(
            jax.ShapeDtypeStruct((T, HIDDEN // 2), jnp.int32),
            jax.ShapeDtypeStruct((T, HIDDEN // 2), jnp.int32),
        ),
        mesh=mesh,
        scratch_types=[
            pltpu.VMEM((TOK_PER_W,), jnp.int32),
            pltpu.VMEM((TOK_PER_W,), jnp.int32),
            pltpu.VMEM((CH, HIDDEN // 2), jnp.int32),
            pltpu.VMEM((CH, HIDDEN // 2), jnp.int32),
            pltpu.VMEM((CH, HIDDEN // 2), jnp.int32),
            pltpu.VMEM((CH, HIDDEN // 2), jnp.int32),
            pltpu.SemaphoreType.DMA,
            pltpu.SemaphoreType.DMA,
            pltpu.SemaphoreType.DMA,
            pltpu.SemaphoreType.DMA,
        ],
    )
    return f(ys, pos1, pos2)


def _add_body(a_ref, b_ref, wa_ref, wb_ref, o_ref):
    a = a_ref.bitcast(jnp.bfloat16)[...].reshape(BLK, HIDDEN).astype(jnp.float32)
    b = b_ref.bitcast(jnp.bfloat16)[...].reshape(BLK, HIDDEN).astype(jnp.float32)
    o_ref[...] = a * wa_ref[:, 0:1] + b * wb_ref[:, 0:1]


def _final_add(y1, y2, w1rep, w2rep):
    return pl.pallas_call(
        _add_body,
        grid=(T // BLK,),
        in_specs=[
            pl.BlockSpec((BLK, HIDDEN // 2), lambda i: (i, 0)),
            pl.BlockSpec((BLK, HIDDEN // 2), lambda i: (i, 0)),
            pl.BlockSpec((BLK, 16), lambda i: (i, 0)),
            pl.BlockSpec((BLK, 16), lambda i: (i, 0)),
        ],
        out_specs=pl.BlockSpec((BLK, HIDDEN), lambda i: (i, 0)),
        out_shape=jax.ShapeDtypeStruct((T, HIDDEN), jnp.float32),
    )(y1, y2, w1rep, w2rep)


def kernel(hidden_states, gate_w, gate_proj, up_proj, down_proj):
    b, s, h = hidden_states.shape
    x = hidden_states.reshape(-1, h)
    pos1c, pos2c, w1rep, w2rep, eid32, xb = _routing(x, gate_w.T)
    pos1 = pos1c.reshape(T)
    pos2 = pos2c.reshape(T)
    xs = _dispatch(xb, pos1, pos2)
    ys = _gmm(eid32, xs, gate_proj, up_proj, down_proj)
    y1, y2 = _combine(ys, pos1, pos2)
    out = _final_add(y1, y2, w1rep, w2rep)
    return out.reshape(b, s, h)


# speedup vs baseline: 1.0339x; 1.0339x over previous
import functools

import jax
import jax.numpy as jnp
from jax import lax
from jax.experimental import pallas as pl
from jax.experimental.pallas import tpu as pltpu
from jax.experimental.pallas import tpu_sc as plsc

HIDDEN = 1024
FFN = 768
E = 8
T = 2048
BLK = 256
NT = (2 * T) // BLK + E - 1
PAD_ROWS = NT * BLK

NC = 2
NS = 16
NW = NC * NS
TOK_PER_W = T // NW
CH = 32
CCH = 256
NCH = T // CCH


def _routing_body(x_ref, gwt_ref, pos1_ref, pos2_ref, w1_ref, w2_ref, eid_ref,
                  xb_ref):
    x = x_ref[...]
    xb_ref.bitcast(jnp.bfloat16)[...] = x.astype(jnp.bfloat16).reshape(2 * T, HIDDEN // 2)
    logits = jnp.dot(x, gwt_ref[...], preferred_element_type=jnp.float32)
    l1 = jnp.max(logits, axis=-1, keepdims=True)
    i1 = jnp.argmax(logits, axis=-1)[:, None]
    cols = jax.lax.broadcasted_iota(jnp.int32, logits.shape, 1)
    l_m = jnp.where(cols == i1, -jnp.inf, logits)
    l2 = jnp.max(l_m, axis=-1, keepdims=True)
    i2 = jnp.argmax(l_m, axis=-1)[:, None]
    w1 = 1.0 / (1.0 + jnp.exp(l2 - l1))
    w2 = 1.0 - w1

    oh1 = cols == i1
    oh2 = cols == i2
    a = (oh1 | oh2).astype(jnp.bfloat16)

    r_i = jax.lax.broadcasted_iota(jnp.int32, (CCH, CCH), 0)
    c_i = jax.lax.broadcasted_iota(jnp.int32, (CCH, CCH), 1)
    tri_c = (c_i < r_i).astype(jnp.bfloat16)
    a3 = a.reshape(NCH, CCH, E)
    within = [
        jnp.dot(tri_c, a3[c], preferred_element_type=jnp.float32)
        for c in range(NCH)
    ]
    s = jnp.stack(
        [jnp.sum(a3[c].astype(jnp.float32), axis=0) for c in range(NCH)], axis=0
    )
    ch_r = jax.lax.broadcasted_iota(jnp.int32, (NCH, NCH), 0)
    ch_c = jax.lax.broadcasted_iota(jnp.int32, (NCH, NCH), 1)
    tri_ch = (ch_c < ch_r).astype(jnp.float32)
    excl_ch = jnp.dot(tri_ch, s, preferred_element_type=jnp.float32)
    p_excl = jnp.concatenate(
        [within[c] + excl_ch[c:c + 1, :] for c in range(NCH)], axis=0
    )

    cnt = jnp.sum(s, axis=0, keepdims=True)
    rup = jnp.ceil(cnt / BLK) * BLK
    e_r = jax.lax.broadcasted_iota(jnp.int32, (E, E), 0)
    e_c = jax.lax.broadcasted_iota(jnp.int32, (E, E), 1)
    tri8 = (e_r < e_c).astype(jnp.float32)
    astart = jnp.dot(rup, tri8, preferred_element_type=jnp.float32)

    posf = astart + p_excl
    pos1 = jnp.sum(jnp.where(oh1, posf, 0.0), axis=-1).astype(jnp.int32)
    pos2 = jnp.sum(jnp.where(oh2, posf, 0.0), axis=-1).astype(jnp.int32)
    pos1_ref[...] = pos1
    pos2_ref[...] = pos2
    w1_ref[...] = jnp.broadcast_to(w1, (T, 16))
    w2_ref[...] = jnp.broadcast_to(w2, (T, 16))

    cum_incl = astart + rup
    tile_start = jax.lax.broadcasted_iota(jnp.int32, (32, E), 0).astype(jnp.float32) * BLK
    eid = jnp.sum((cum_incl <= tile_start).astype(jnp.int32), axis=-1)
    n_active = (jnp.sum(rup) / BLK).astype(jnp.int32)
    e_iota = jax.lax.broadcasted_iota(jnp.int32, (1, E), 1)
    last_e = jnp.max(jnp.where(rup > 0, e_iota, 0))
    idx32 = jax.lax.broadcasted_iota(jnp.int32, (32,), 0)
    eid_ref[...] = jnp.where(idx32 == NT, n_active, jnp.minimum(eid, last_e))


def _routing(x, gate_wt):
    return pl.pallas_call(
        _routing_body,
        out_shape=(
            jax.ShapeDtypeStruct((T,), jnp.int32),
            jax.ShapeDtypeStruct((T,), jnp.int32),
            jax.ShapeDtypeStruct((T, 16), jnp.float32),
            jax.ShapeDtypeStruct((T, 16), jnp.float32),
            jax.ShapeDtypeStruct((32,), jnp.int32),
            jax.ShapeDtypeStruct((T, HIDDEN // 2), jnp.int32),
        ),
    )(x, gate_wt)


def _dispatch_body(xb_hbm, p1_hbm, p2_hbm, xs_hbm, xv, i1v, i2v, s1, s2):
    wid = lax.axis_index("s") * NC + lax.axis_index("c")
    base = wid * TOK_PER_W
    pltpu.sync_copy(xb_hbm.at[pl.ds(base, TOK_PER_W)], xv)
    pltpu.sync_copy(p1_hbm.at[pl.ds(base, TOK_PER_W)], i1v)
    pltpu.sync_copy(p2_hbm.at[pl.ds(base, TOK_PER_W)], i2v)
    c1 = pltpu.async_copy(xv, xs_hbm.at[i1v], s1)
    c2 = pltpu.async_copy(xv, xs_hbm.at[i2v], s2)
    c1.wait()
    c2.wait()


def _dispatch(xb, pos1, pos2):
    mesh = plsc.VectorSubcoreMesh(core_axis_name="c", subcore_axis_name="s")
    f = pl.kernel(
        _dispatch_body,
        out_type=jax.ShapeDtypeStruct((PAD_ROWS, HIDDEN // 2), jnp.int32),
        mesh=mesh,
        scratch_types=[
            pltpu.VMEM((TOK_PER_W, HIDDEN // 2), jnp.int32),
            pltpu.VMEM((TOK_PER_W,), jnp.int32),
            pltpu.VMEM((TOK_PER_W,), jnp.int32),
            pltpu.SemaphoreType.DMA,
            pltpu.SemaphoreType.DMA,
        ],
    )
    return f(xb, pos1, pos2)


def _gmm_body(eids_ref, xs_ref, g_ref, u_ref, d_ref, out_ref, gb_ref, ub_ref, db_ref):
    i = pl.program_id(0)
    n_active = eids_ref[NT]
    changed = jnp.logical_or(i == 0, eids_ref[i] != eids_ref[jnp.maximum(i - 1, 0)])

    @pl.when(jnp.logical_and(i < n_active, changed))
    def _():
        gb_ref[...] = g_ref[0].astype(jnp.bfloat16)
        ub_ref[...] = u_ref[0].astype(jnp.bfloat16)
        db_ref[...] = d_ref[0].astype(jnp.bfloat16)

    @pl.when(i < n_active)
    def _():
        x = xs_ref.bitcast(jnp.bfloat16)[...].reshape(BLK, HIDDEN)
        g = jnp.dot(x, gb_ref[...], preferred_element_type=jnp.float32)
        u = jnp.dot(x, ub_ref[...], preferred_element_type=jnp.float32)
        act = ((g * jax.nn.sigmoid(g)) * u).astype(jnp.bfloat16)
        y = jnp.dot(act, db_ref[...], preferred_element_type=jnp.float32)
        out_ref.bitcast(jnp.bfloat16)[...] = y.astype(jnp.bfloat16).reshape(2 * BLK, HIDDEN // 2)


def _gmm(eids, xs, gate_proj, up_proj, down_proj):
    grid_spec = pltpu.PrefetchScalarGridSpec(
        num_scalar_prefetch=1,
        grid=(NT,),
        in_specs=[
            pl.BlockSpec((BLK, HIDDEN // 2),
                         lambda i, eids: (jnp.minimum(i, eids[NT] - 1), 0)),
            pl.BlockSpec((1, HIDDEN, FFN), lambda i, eids: (eids[i], 0, 0)),
            pl.BlockSpec((1, HIDDEN, FFN), lambda i, eids: (eids[i], 0, 0)),
            pl.BlockSpec((1, FFN, HIDDEN), lambda i, eids: (eids[i], 0, 0)),
        ],
        out_specs=pl.BlockSpec((BLK, HIDDEN // 2),
                               lambda i, eids: (jnp.minimum(i, eids[NT] - 1), 0)),
        scratch_shapes=[
            pltpu.VMEM((HIDDEN, FFN), jnp.bfloat16),
            pltpu.VMEM((HIDDEN, FFN), jnp.bfloat16),
            pltpu.VMEM((FFN, HIDDEN), jnp.bfloat16),
        ],
    )
    return pl.pallas_call(
        _gmm_body,
        grid_spec=grid_spec,
        out_shape=jax.ShapeDtypeStruct((PAD_ROWS, HIDDEN // 2), jnp.int32),
        compiler_params=pltpu.CompilerParams(
            dimension_semantics=("arbitrary",),
        ),
    )(eids, xs, gate_proj, up_proj, down_proj)


def _combine_body(ys_hbm, p1_hbm, p2_hbm, y1_hbm, y2_hbm, i1v, i2v,
                  b10, b11, b20, b21, s10, s11, s20, s21):
    wid = lax.axis_index("s") * NC + lax.axis_index("c")
    base = wid * TOK_PER_W
    pltpu.sync_copy(p1_hbm.at[pl.ds(base, TOK_PER_W)], i1v)
    pltpu.sync_copy(p2_hbm.at[pl.ds(base, TOK_PER_W)], i2v)
    bufs1, bufs2 = (b10, b11), (b20, b21)
    sems1, sems2 = (s10, s11), (s20, s21)
    cps = []
    for c in range(TOK_PER_W // CH):
        cps.append((
            pltpu.async_copy(ys_hbm.at[i1v.at[pl.ds(c * CH, CH)]], bufs1[c], sems1[c]),
            pltpu.async_copy(ys_hbm.at[i2v.at[pl.ds(c * CH, CH)]], bufs2[c], sems2[c]),
        ))
    for c in range(TOK_PER_W // CH):
        g1, g2 = cps[c]
        g1.wait()
        pltpu.sync_copy(bufs1[c], y1_hbm.at[pl.ds(base + c * CH, CH)])
        g2.wait()
        pltpu.sync_copy(bufs2[c], y2_hbm.at[pl.ds(base + c * CH, CH)])


def _combine(ys, pos1, pos2):
    mesh = plsc.VectorSubcoreMesh(core_axis_name="c", subcore_axis_name="s")
    f = pl.kernel(
        _combine_body,
        out_type=(
            jax.ShapeDtypeStruct((T, HIDDEN // 2), jnp.int32),
            jax.ShapeDtypeStruct((T, HIDDEN // 2), jnp.int32),
        ),
        mesh=mesh,
        scratch_types=[
            pltpu.VMEM((TOK_PER_W,), jnp.int32),
            pltpu.VMEM((TOK_PER_W,), jnp.int32),
            pltpu.VMEM((CH, HIDDEN // 2), jnp.int32),
            pltpu.VMEM((CH, HIDDEN // 2), jnp.int32),
            pltpu.VMEM((CH, HIDDEN // 2), jnp.int32),
            pltpu.VMEM((CH, HIDDEN // 2), jnp.int32),
            pltpu.SemaphoreType.DMA,
            pltpu.SemaphoreType.DMA,
            pltpu.SemaphoreType.DMA,
            pltpu.SemaphoreType.DMA,
        ],
    )
    return f(ys, pos1, pos2)


def _add_body(a_ref, b_ref, wa_ref, wb_ref, o_ref):
    a = a_ref.bitcast(jnp.bfloat16)[...].reshape(BLK, HIDDEN).astype(jnp.float32)
    b = b_ref.bitcast(jnp.bfloat16)[...].reshape(BLK, HIDDEN).astype(jnp.float32)
    o_ref[...] = a * wa_ref[:, 0:1] + b * wb_ref[:, 0:1]


def _final_add(y1, y2, w1rep, w2rep):
    return pl.pallas_call(
        _add_body,
        grid=(T // BLK,),
        in_specs=[
            pl.BlockSpec((BLK, HIDDEN // 2), lambda i: (i, 0)),
            pl.BlockSpec((BLK, HIDDEN // 2), lambda i: (i, 0)),
            pl.BlockSpec((BLK, 16), lambda i: (i, 0)),
            pl.BlockSpec((BLK, 16), lambda i: (i, 0)),
        ],
        out_specs=pl.BlockSpec((BLK, HIDDEN), lambda i: (i, 0)),
        out_shape=jax.ShapeDtypeStruct((T, HIDDEN), jnp.float32),
    )(y1, y2, w1rep, w2rep)


def kernel(hidden_states, gate_w, gate_proj, up_proj, down_proj):
    b, s, h = hidden_states.shape
    x = hidden_states.reshape(-1, h)
    pos1, pos2, w1rep, w2rep, eid32, xb = _routing(x, gate_w.T)
    xs = _dispatch(xb, pos1, pos2)
    ys = _gmm(eid32, xs, gate_proj, up_proj, down_proj)
    y1, y2 = _combine(ys, pos1, pos2)
    out = _final_add(y1, y2, w1rep, w2rep)
    return out.reshape(b, s, h)


# speedup vs baseline: 1.0478x; 1.0135x over previous
import functools

import jax
import jax.numpy as jnp
from jax import lax
from jax.experimental import pallas as pl
from jax.experimental.pallas import tpu as pltpu
from jax.experimental.pallas import tpu_sc as plsc

HIDDEN = 1024
FFN = 768
E = 8
T = 2048
BLK = 256
NT = (2 * T) // BLK + E - 1
PAD_ROWS = NT * BLK

NC = 2
NS = 16
NW = NC * NS
TOK_PER_W = T // NW
CH = 32
CCH = 256
NCH = T // CCH


def _routing_body(x_ref, gwt_ref, pos1_ref, pos2_ref, w1_ref, w2_ref, eid_ref,
                  xb_ref):
    x = x_ref[...]
    xb_ref.bitcast(jnp.bfloat16)[...] = x.astype(jnp.bfloat16).reshape(2 * T, HIDDEN // 2)
    logits = jnp.dot(x, gwt_ref[...], preferred_element_type=jnp.float32)
    l1 = jnp.max(logits, axis=-1, keepdims=True)
    i1 = jnp.argmax(logits, axis=-1)[:, None]
    cols = jax.lax.broadcasted_iota(jnp.int32, logits.shape, 1)
    l_m = jnp.where(cols == i1, -jnp.inf, logits)
    l2 = jnp.max(l_m, axis=-1, keepdims=True)
    i2 = jnp.argmax(l_m, axis=-1)[:, None]
    w1 = 1.0 / (1.0 + jnp.exp(l2 - l1))
    w2 = 1.0 - w1

    oh1 = cols == i1
    oh2 = cols == i2
    a = (oh1 | oh2).astype(jnp.bfloat16)

    r_i = jax.lax.broadcasted_iota(jnp.int32, (CCH, CCH), 0)
    c_i = jax.lax.broadcasted_iota(jnp.int32, (CCH, CCH), 1)
    tri_c = (c_i < r_i).astype(jnp.bfloat16)
    a3 = a.reshape(NCH, CCH, E)
    within = [
        jnp.dot(tri_c, a3[c], preferred_element_type=jnp.float32)
        for c in range(NCH)
    ]
    s = jnp.stack(
        [jnp.sum(a3[c].astype(jnp.float32), axis=0) for c in range(NCH)], axis=0
    )
    ch_r = jax.lax.broadcasted_iota(jnp.int32, (NCH, NCH), 0)
    ch_c = jax.lax.broadcasted_iota(jnp.int32, (NCH, NCH), 1)
    tri_ch = (ch_c < ch_r).astype(jnp.float32)
    excl_ch = jnp.dot(tri_ch, s, preferred_element_type=jnp.float32)
    p_excl = jnp.concatenate(
        [within[c] + excl_ch[c:c + 1, :] for c in range(NCH)], axis=0
    )

    cnt = jnp.sum(s, axis=0, keepdims=True)
    rup = jnp.ceil(cnt / BLK) * BLK
    e_r = jax.lax.broadcasted_iota(jnp.int32, (E, E), 0)
    e_c = jax.lax.broadcasted_iota(jnp.int32, (E, E), 1)
    tri8 = (e_r < e_c).astype(jnp.float32)
    astart = jnp.dot(rup, tri8, preferred_element_type=jnp.float32)

    posf = astart + p_excl
    pos1 = jnp.sum(jnp.where(oh1, posf, 0.0), axis=-1).astype(jnp.int32)
    pos2 = jnp.sum(jnp.where(oh2, posf, 0.0), axis=-1).astype(jnp.int32)
    pos1_ref[...] = pos1
    pos2_ref[...] = pos2
    w1_ref[...] = jnp.broadcast_to(w1, (T, 16))
    w2_ref[...] = jnp.broadcast_to(w2, (T, 16))

    cum_incl = astart + rup
    tile_start = jax.lax.broadcasted_iota(jnp.int32, (32, E), 0).astype(jnp.float32) * BLK
    eid = jnp.sum((cum_incl <= tile_start).astype(jnp.int32), axis=-1)
    n_active = (jnp.sum(rup) / BLK).astype(jnp.int32)
    e_iota = jax.lax.broadcasted_iota(jnp.int32, (1, E), 1)
    last_e = jnp.max(jnp.where(rup > 0, e_iota, 0))
    idx32 = jax.lax.broadcasted_iota(jnp.int32, (32,), 0)
    eid_ref[...] = jnp.where(idx32 == NT, n_active, jnp.minimum(eid, last_e))


def _routing(x, gate_wt):
    return pl.pallas_call(
        _routing_body,
        out_shape=(
            jax.ShapeDtypeStruct((T,), jnp.int32),
            jax.ShapeDtypeStruct((T,), jnp.int32),
            jax.ShapeDtypeStruct((T, 16), jnp.float32),
            jax.ShapeDtypeStruct((T, 16), jnp.float32),
            jax.ShapeDtypeStruct((32,), jnp.int32),
            jax.ShapeDtypeStruct((T, HIDDEN // 2), jnp.int32),
        ),
    )(x, gate_wt)


def _dispatch_body(xb_hbm, p1_hbm, p2_hbm, xs_hbm, xv, i1v, i2v, s1, s2, s3):
    wid = lax.axis_index("s") * NC + lax.axis_index("c")
    base = wid * TOK_PER_W
    l1 = pltpu.async_copy(xb_hbm.at[pl.ds(base, TOK_PER_W)], xv, s1)
    l2 = pltpu.async_copy(p1_hbm.at[pl.ds(base, TOK_PER_W)], i1v, s2)
    l3 = pltpu.async_copy(p2_hbm.at[pl.ds(base, TOK_PER_W)], i2v, s3)
    l1.wait()
    l2.wait()
    l3.wait()
    c1 = pltpu.async_copy(xv, xs_hbm.at[i1v], s1)
    c2 = pltpu.async_copy(xv, xs_hbm.at[i2v], s2)
    c1.wait()
    c2.wait()


def _dispatch(xb, pos1, pos2):
    mesh = plsc.VectorSubcoreMesh(core_axis_name="c", subcore_axis_name="s")
    f = pl.kernel(
        _dispatch_body,
        out_type=jax.ShapeDtypeStruct((PAD_ROWS, HIDDEN // 2), jnp.int32),
        mesh=mesh,
        scratch_types=[
            pltpu.VMEM((TOK_PER_W, HIDDEN // 2), jnp.int32),
            pltpu.VMEM((TOK_PER_W,), jnp.int32),
            pltpu.VMEM((TOK_PER_W,), jnp.int32),
            pltpu.SemaphoreType.DMA,
            pltpu.SemaphoreType.DMA,
            pltpu.SemaphoreType.DMA,
        ],
    )
    return f(xb, pos1, pos2)


def _gmm_body(eids_ref, xs_ref, g_ref, u_ref, d_ref, out_ref, gb_ref, ub_ref, db_ref):
    i = pl.program_id(0)
    n_active = eids_ref[NT]
    changed = jnp.logical_or(i == 0, eids_ref[i] != eids_ref[jnp.maximum(i - 1, 0)])

    @pl.when(jnp.logical_and(i < n_active, changed))
    def _():
        gb_ref[...] = g_ref[0].astype(jnp.bfloat16)
        ub_ref[...] = u_ref[0].astype(jnp.bfloat16)
        db_ref[...] = d_ref[0].astype(jnp.bfloat16)

    @pl.when(i < n_active)
    def _():
        x = xs_ref.bitcast(jnp.bfloat16)[...].reshape(BLK, HIDDEN)
        g = jnp.dot(x, gb_ref[...], preferred_element_type=jnp.float32)
        u = jnp.dot(x, ub_ref[...], preferred_element_type=jnp.float32)
        act = ((g * jax.nn.sigmoid(g)) * u).astype(jnp.bfloat16)
        y = jnp.dot(act, db_ref[...], preferred_element_type=jnp.float32)
        out_ref.bitcast(jnp.bfloat16)[...] = y.astype(jnp.bfloat16).reshape(2 * BLK, HIDDEN // 2)


def _gmm(eids, xs, gate_proj, up_proj, down_proj):
    grid_spec = pltpu.PrefetchScalarGridSpec(
        num_scalar_prefetch=1,
        grid=(NT,),
        in_specs=[
            pl.BlockSpec((BLK, HIDDEN // 2),
                         lambda i, eids: (jnp.minimum(i, eids[NT] - 1), 0)),
            pl.BlockSpec((1, HIDDEN, FFN), lambda i, eids: (eids[i], 0, 0)),
            pl.BlockSpec((1, HIDDEN, FFN), lambda i, eids: (eids[i], 0, 0)),
            pl.BlockSpec((1, FFN, HIDDEN), lambda i, eids: (eids[i], 0, 0)),
        ],
        out_specs=pl.BlockSpec((BLK, HIDDEN // 2),
                               lambda i, eids: (jnp.minimum(i, eids[NT] - 1), 0)),
        scratch_shapes=[
            pltpu.VMEM((HIDDEN, FFN), jnp.bfloat16),
            pltpu.VMEM((HIDDEN, FFN), jnp.bfloat16),
            pltpu.VMEM((FFN, HIDDEN), jnp.bfloat16),
        ],
    )
    return pl.pallas_call(
        _gmm_body,
        grid_spec=grid_spec,
        out_shape=jax.ShapeDtypeStruct((PAD_ROWS, HIDDEN // 2), jnp.int32),
        compiler_params=pltpu.CompilerParams(
            dimension_semantics=("arbitrary",),
        ),
    )(eids, xs, gate_proj, up_proj, down_proj)


def _combine_body(ys_hbm, p1_hbm, p2_hbm, y1_hbm, y2_hbm, i1v, i2v,
                  b10, b11, b20, b21, s10, s11, s20, s21):
    wid = lax.axis_index("s") * NC + lax.axis_index("c")
    base = wid * TOK_PER_W
    l1 = pltpu.async_copy(p1_hbm.at[pl.ds(base, TOK_PER_W)], i1v, s10)
    l2 = pltpu.async_copy(p2_hbm.at[pl.ds(base, TOK_PER_W)], i2v, s20)
    l1.wait()
    l2.wait()
    bufs1, bufs2 = (b10, b11), (b20, b21)
    sems1, sems2 = (s10, s11), (s20, s21)
    cps = []
    for c in range(TOK_PER_W // CH):
        cps.append((
            pltpu.async_copy(ys_hbm.at[i1v.at[pl.ds(c * CH, CH)]], bufs1[c], sems1[c]),
            pltpu.async_copy(ys_hbm.at[i2v.at[pl.ds(c * CH, CH)]], bufs2[c], sems2[c]),
        ))
    for c in range(TOK_PER_W // CH):
        g1, g2 = cps[c]
        g1.wait()
        pltpu.sync_copy(bufs1[c], y1_hbm.at[pl.ds(base + c * CH, CH)])
        g2.wait()
        pltpu.sync_copy(bufs2[c], y2_hbm.at[pl.ds(base + c * CH, CH)])


def _combine(ys, pos1, pos2):
    mesh = plsc.VectorSubcoreMesh(core_axis_name="c", subcore_axis_name="s")
    f = pl.kernel(
        _combine_body,
        out_type=(
            jax.ShapeDtypeStruct((T, HIDDEN // 2), jnp.int32),
            jax.ShapeDtypeStruct((T, HIDDEN // 2), jnp.int32),
        ),
        mesh=mesh,
        scratch_types=[
            pltpu.VMEM((TOK_PER_W,), jnp.int32),
            pltpu.VMEM((TOK_PER_W,), jnp.int32),
            pltpu.VMEM((CH, HIDDEN // 2), jnp.int32),
            pltpu.VMEM((CH, HIDDEN // 2), jnp.int32),
            pltpu.VMEM((CH, HIDDEN // 2), jnp.int32),
            pltpu.VMEM((CH, HIDDEN // 2), jnp.int32),
            pltpu.SemaphoreType.DMA,
            pltpu.SemaphoreType.DMA,
            pltpu.SemaphoreType.DMA,
            pltpu.SemaphoreType.DMA,
        ],
    )
    return f(ys, pos1, pos2)


def _add_body(a_ref, b_ref, wa_ref, wb_ref, o_ref):
    a = a_ref.bitcast(jnp.bfloat16)[...].reshape(BLK, HIDDEN).astype(jnp.float32)
    b = b_ref.bitcast(jnp.bfloat16)[...].reshape(BLK, HIDDEN).astype(jnp.float32)
    o_ref[...] = a * wa_ref[:, 0:1] + b * wb_ref[:, 0:1]


def _final_add(y1, y2, w1rep, w2rep):
    return pl.pallas_call(
        _add_body,
        grid=(T // BLK,),
        in_specs=[
            pl.BlockSpec((BLK, HIDDEN // 2), lambda i: (i, 0)),
            pl.BlockSpec((BLK, HIDDEN // 2), lambda i: (i, 0)),
            pl.BlockSpec((BLK, 16), lambda i: (i, 0)),
            pl.BlockSpec((BLK, 16), lambda i: (i, 0)),
        ],
        out_specs=pl.BlockSpec((BLK, HIDDEN), lambda i: (i, 0)),
        out_shape=jax.ShapeDtypeStruct((T, HIDDEN), jnp.float32),
    )(y1, y2, w1rep, w2rep)


def kernel(hidden_states, gate_w, gate_proj, up_proj, down_proj):
    b, s, h = hidden_states.shape
    x = hidden_states.reshape(-1, h)
    pos1, pos2, w1rep, w2rep, eid32, xb = _routing(x, gate_w.T)
    xs = _dispatch(xb, pos1, pos2)
    ys = _gmm(eid32, xs, gate_proj, up_proj, down_proj)
    y1, y2 = _combine(ys, pos1, pos2)
    out = _final_add(y1, y2, w1rep, w2rep)
    return out.reshape(b, s, h)
